# Initial kernel scaffold; baseline (speedup 1.0000x reference)
import jax
import jax.numpy as jnp
from jax.experimental import pallas as pl


def kernel(features, wg_graph, We_graph, be_graph, wg_motif, We_motif, be_motif, wg_node, We_node, be_node):
    raise NotImplementedError("write your pallas kernel here")


# speedup vs baseline: 4.3279x; 4.3279x over previous
import functools

import jax
import jax.numpy as jnp
from jax.experimental import pallas as pl
from jax.experimental.pallas import tpu as pltpu

E = 3
B, S, H = 4, 2048, 768
T = B * S
C = -(-T // E)
N = 512
NB = T // N


def _cumsum_sublane(a, n):
    d = 1
    while d < n:
        shifted = jnp.concatenate(
            [jnp.zeros((d,) + a.shape[1:], a.dtype), a[:-d]], axis=0)
        a = a + shifted
        d *= 2
    return a


def _moe_body(x_ref, wg_ref, We_ref, be_ref, out_ref, counts_ref):
    j = pl.program_id(1)

    @pl.when(j == 0)
    def _():
        counts_ref[0] = 0.0
        counts_ref[1] = 0.0
        counts_ref[2] = 0.0

    x = x_ref[0, 0]
    wg = wg_ref[0]
    logits = jnp.dot(x, wg, preferred_element_type=jnp.float32)
    l0 = logits[:, 0:1]
    l1 = logits[:, 1:2]
    l2 = logits[:, 2:3]
    m = jnp.maximum(jnp.maximum(l0, l1), l2)
    denom = jnp.exp(l0 - m) + jnp.exp(l1 - m) + jnp.exp(l2 - m)
    gate_top = 1.0 / denom

    b0 = l0 == m
    b1 = (l1 == m) & ~b0
    b2 = (l2 == m) & ~b0 & ~b1
    f0 = b0.astype(jnp.float32)
    f1 = b1.astype(jnp.float32)
    f2 = b2.astype(jnp.float32)

    c0 = _cumsum_sublane(f0, N)
    c1 = _cumsum_sublane(f1, N)
    c2 = _cumsum_sublane(f2, N)

    n0 = counts_ref[0]
    n1 = counts_ref[1]
    n2 = counts_ref[2]
    cap = float(C)
    keep = (f0 * (c0 - 1.0 + n0 < cap)
            + f1 * (c1 - 1.0 + n1 < cap)
            + f2 * (c2 - 1.0 + n2 < cap))
    counts_ref[0] = n0 + jnp.sum(f0)
    counts_ref[1] = n1 + jnp.sum(f1)
    counts_ref[2] = n2 + jnp.sum(f2)

    gate = gate_top * keep

    y = (jnp.dot(x * f0, We_ref[0, 0], preferred_element_type=jnp.float32)
         + jnp.dot(x * f1, We_ref[0, 1], preferred_element_type=jnp.float32)
         + jnp.dot(x * f2, We_ref[0, 2], preferred_element_type=jnp.float32))
    bias = (f0 * be_ref[0, 0:1, :] + f1 * be_ref[0, 1:2, :]
            + f2 * be_ref[0, 2:3, :])
    out_ref[0, 0] = (y + bias) * gate


@functools.partial(jax.jit, static_argnames=("interpret",))
def _moe_all(features, wg_all, We_all, be_all, interpret=False):
    grid = (3, NB)
    sb = S // N
    return pl.pallas_call(
        _moe_body,
        grid=grid,
        in_specs=[
            pl.BlockSpec((1, 1, N, H), lambda k, j: (j // sb, k, j % sb, 0)),
            pl.BlockSpec((1, H, 128), lambda k, j: (k, 0, 0)),
            pl.BlockSpec((1, E, H, H), lambda k, j: (k, 0, 0, 0)),
            pl.BlockSpec((1, E, H), lambda k, j: (k, 0, 0)),
        ],
        out_specs=pl.BlockSpec((1, 1, N, H), lambda k, j: (j // sb, k, j % sb, 0)),
        out_shape=jax.ShapeDtypeStruct((B, 3, S, H), jnp.float32),
        scratch_shapes=[pltpu.SMEM((3,), jnp.float32)],
        interpret=interpret,
    )(features, wg_all, We_all, be_all)


def kernel(features, wg_graph, We_graph, be_graph, wg_motif, We_motif,
           be_motif, wg_node, We_node, be_node, interpret=False):
    wg_all = jnp.stack([wg_graph, wg_motif, wg_node])
    wg_all = jnp.pad(wg_all, ((0, 0), (0, 0), (0, 128 - E)))
    We_all = jnp.stack([We_graph, We_motif, We_node])
    be_all = jnp.stack([be_graph, be_motif, be_node])
    return _moe_all(features, wg_all, We_all, be_all, interpret=interpret)
